# baseline (speedup 1.0000x reference)
import functools

import jax
import jax.numpy as jnp
from jax import lax
from jax.experimental import pallas as pl
from jax.experimental.pallas import tpu as pltpu
from jax.experimental.pallas import tpu_sc as plsc

TAU = 5.0
B = 16384
C = 100
CP = 128
V = 100000
NW = 32
RPW = 3128
NSEL = 768
CHR = 256
CCAP = 128
BLK = 1024
GRID = B // BLK
LANES = 16
NWIN = NSEL // LANES


def _tc_body(logits_ref, y_ref, probs_ref, loss_ref):
    x = logits_ref[...]
    m = jnp.max(x, axis=1, keepdims=True)
    xm = x - m
    e1 = jnp.exp(xm)
    s1 = jnp.sum(e1, axis=1)
    e5 = jnp.exp(xm * (1.0 / TAU))
    s5 = jnp.sum(e5, axis=1, keepdims=True)
    probs_ref[...] = e5 / s5
    y = y_ref[0, 0, :]
    col = lax.broadcasted_iota(jnp.int32, (BLK, CP), 1)
    xy = jnp.sum(jnp.where(col == y[:, None], xm, 0.0), axis=1)
    part = jnp.sum(jnp.log(s1) - xy) * (1.0 / B)

    @pl.when(pl.program_id(0) == 0)
    def _init():
        loss_ref[0, 0] = 0.0

    loss_ref[0, 0] += part


def _tc_probs_loss(logits_pad, y3):
    return pl.pallas_call(
        _tc_body,
        grid=(GRID,),
        in_specs=[
            pl.BlockSpec((BLK, CP), lambda i: (i, 0)),
            pl.BlockSpec((1, 1, BLK), lambda i: (i, 0, 0)),
        ],
        out_specs=[
            pl.BlockSpec((BLK, CP), lambda i: (i, 0)),
            pl.BlockSpec(memory_space=pltpu.SMEM),
        ],
        out_shape=[
            jax.ShapeDtypeStruct((B, CP), jnp.float32),
            jax.ShapeDtypeStruct((1, 1), jnp.float32),
        ],
    )(logits_pad, y3)


_SHUF_DNUMS = lax.GatherDimensionNumbers(
    offset_dims=(), collapsed_slice_dims=(0,), start_index_map=(0,))


def _shuf(x, ix):
    return lax.gather(x, ix[:, None], _SHUF_DNUMS, (1,),
                      mode=lax.GatherScatterMode.PROMISE_IN_BOUNDS)


def _compact(lane, m, val):
    p = jnp.where(m, 1, 0)
    for kk in (1, 2, 4, 8):
        p = p + jnp.where(lane >= kk,
                          _shuf(p, jnp.maximum(lane - kk, 0)), 0)
    s = jnp.zeros((LANES,), jnp.int32)
    for step in (8, 4, 2, 1):
        t = s + step
        pv = _shuf(p, jnp.minimum(t - 1, LANES - 1))
        s = jnp.where(pv < lane + 1, t, s)
    comp = _shuf(val, jnp.minimum(s, LANES - 1))
    return comp, p[LANES - 1]


_COLS = (0, 16, 32, 48, 64, 80, 84)


def _sc_body(bidx_hbm, probs_hbm, out_hbm,
             idxbuf, posflat, chunkid, chunklist, poslist, staging,
             chunkbuf, semg):
    nc = 2
    wid = lax.axis_index("s") * nc + lax.axis_index("c")
    base = wid * RPW
    lane = lax.iota(jnp.int32, LANES)
    is_last = wid == NW - 1

    pltpu.sync_copy(bidx_hbm, idxbuf)
    lo = base
    hi = jnp.minimum(base + RPW, V)

    def scan_step(c, cur):
        v = idxbuf[pl.ds(c * LANES, LANES)]
        m = (v >= lo) & (v < hi)
        val = (v << 14) | (c * LANES + lane)
        comp, cnt = _compact(lane, m, val)
        posflat[pl.ds(jnp.minimum(cur, NSEL - LANES), LANES)] = comp
        return cur + cnt

    cursor = lax.fori_loop(0, B // LANES, scan_step, 0)
    nwin = jnp.minimum((cursor + LANES - 1) // LANES, NWIN)

    for k in range(NWIN):
        wv = posflat[pl.ds(k * LANES, LANES)]
        chunkid[pl.ds(k * LANES, LANES)] = ((wv >> 14) - base) >> 8

    zero16 = jnp.zeros((LANES,), jnp.float32)
    for r in range(CHR):
        for cc in _COLS:
            chunkbuf[r, pl.ds(cc, LANES)] = zero16

    def do_chunk(k, rows):
        cs = base + k * CHR
        csp = cs << 14

        def filt(w, cur):
            wv = posflat[pl.ds(w * LANES, LANES)]
            cid = chunkid[pl.ds(w * LANES, LANES)]
            m = ((w * LANES + lane) < cursor) & (cid == k)
            comp, cnt = _compact(lane, m, wv - csp)
            chunklist[pl.ds(jnp.minimum(cur, CCAP - LANES), LANES)] = comp
            return cur + cnt

        cnt_k = lax.fori_loop(0, nwin, filt, 0)
        cnt_k = jnp.minimum(cnt_k, CCAP)

        for w in range(CCAP // LANES):
            lw = chunklist[pl.ds(w * LANES, LANES)]
            keep = (w * LANES + lane) < cnt_k
            poslist[pl.ds(w * LANES, LANES)] = jnp.where(
                keep, lw & (2**14 - 1), 0)

        @pl.when(cnt_k > 0)
        def _g():
            pltpu.async_copy(probs_hbm.at[poslist], staging, semg).wait()

        def slot_of(t):
            w = t // LANES
            lw = chunklist[pl.ds(w * LANES, LANES)]
            rot = _shuf(lw, (lane + (t - w * LANES)) & (LANES - 1))
            return rot[0] >> 14

        def place(t, _):
            slot = slot_of(t)
            for cc in _COLS:
                chunkbuf[slot, pl.ds(cc, LANES)] = \
                    staging[t, pl.ds(cc, LANES)]
            return 0

        lax.fori_loop(0, cnt_k, place, 0)
        pltpu.sync_copy(chunkbuf.at[pl.ds(0, rows)],
                        out_hbm.at[pl.ds(cs, rows)])

        def wipe(t, _):
            slot = slot_of(t)
            for cc in _COLS:
                chunkbuf[slot, pl.ds(cc, LANES)] = zero16
            return 0

        lax.fori_loop(0, cnt_k, wipe, 0)

    for k in range(11):
        do_chunk(k, CHR)

    @pl.when(jnp.logical_not(is_last))
    def _tail_main():
        do_chunk(11, CHR)
        do_chunk(12, 56)

    @pl.when(is_last)
    def _tail_last():
        do_chunk(11, 216)


@functools.cache
def _sc_scatter():
    return functools.partial(
        pl.kernel,
        out_type=jax.ShapeDtypeStruct((V, C), jnp.float32),
        mesh=plsc.VectorSubcoreMesh(core_axis_name="c",
                                    subcore_axis_name="s"),
        compiler_params=pltpu.CompilerParams(use_tc_tiling_on_sc=True),
        scratch_types=[
            pltpu.VMEM((B,), jnp.int32),
            pltpu.VMEM((NSEL,), jnp.int32),
            pltpu.VMEM((NSEL,), jnp.int32),
            pltpu.VMEM((CCAP,), jnp.int32),
            pltpu.VMEM((CCAP,), jnp.int32),
            pltpu.VMEM((CCAP, CP), jnp.float32),
            pltpu.VMEM((CHR, C), jnp.float32),
            pltpu.SemaphoreType.DMA,
        ],
    )(_sc_body)


def kernel(batch_idx, logits, y_true, labels):
    y3 = y_true.astype(jnp.int32).reshape(GRID, 1, BLK)
    logits_pad = jnp.pad(logits, ((0, 0), (0, CP - C)),
                         constant_values=-1e30)
    probs_pad, loss = _tc_probs_loss(logits_pad, y3)
    labels_new = _sc_scatter()(batch_idx.astype(jnp.int32), probs_pad)
    return (loss.reshape(()), labels_new)
